# Initial kernel scaffold; baseline (speedup 1.0000x reference)
import jax
import jax.numpy as jnp
from jax.experimental import pallas as pl


def kernel(x, W_input, W_recurrent):
    raise NotImplementedError("write your pallas kernel here")


# speedup vs baseline: 10.5804x; 10.5804x over previous
import functools

import jax
import jax.numpy as jnp
from jax.experimental import pallas as pl
from jax.experimental.pallas import tpu as pltpu


def _encoder_kernel(x_ref, wi_ref, wr_ref, out_ref, u_ref, *, steps, batch,
                    rec, kk):
    u_ref[:] = jnp.dot(x_ref[:], wi_ref[:], preferred_element_type=jnp.float32)
    wr = wr_ref[:]

    def step(t, r):
        z = jnp.dot(r, wr, preferred_element_type=jnp.float32)
        bits = jax.lax.bitcast_convert_type(z, jnp.uint32)
        ukey = jnp.where(z < 0, ~bits, bits | jnp.uint32(0x80000000))
        prefix = jnp.zeros((batch, 1), jnp.uint32)
        for b in range(31, -1, -1):
            cand = prefix | jnp.uint32(1 << b)
            cnt = jnp.sum((ukey >= cand).astype(jnp.int32), axis=1,
                          keepdims=True)
            prefix = jnp.where(cnt >= kk, cand, prefix)
        s = jnp.where(ukey >= prefix, z, 0.0)
        a = jnp.tanh(u_ref[pl.ds(t * batch, batch), :] + s)
        nrm = jnp.sqrt(jnp.sum(a * a, axis=1, keepdims=True)) + 1e-6
        return a / nrm

    r0 = jnp.zeros((batch, rec), jnp.float32)
    out_ref[:] = jax.lax.fori_loop(0, steps, step, r0, unroll=False)


def kernel(x, W_input, W_recurrent):
    B, T, H, W, E = x.shape
    R = T * H * W
    rec = W_recurrent.shape[0]
    kk = int(rec * 0.4)
    x2 = jnp.transpose(x.reshape(B, R, E), (1, 0, 2)).reshape(R * B, E)
    return pl.pallas_call(
        functools.partial(_encoder_kernel, steps=R, batch=B, rec=rec, kk=kk),
        out_shape=jax.ShapeDtypeStruct((B, rec), x.dtype),
        scratch_shapes=[pltpu.VMEM((R * B, rec), jnp.float32)],
    )(x2, W_input, W_recurrent)


# speedup vs baseline: 18.7410x; 1.7713x over previous
import functools

import jax
import jax.numpy as jnp
from jax.experimental import pallas as pl
from jax.experimental.pallas import tpu as pltpu


def _kth_threshold(ukey, kf, batch):
    prefix = jnp.zeros((batch, 1), jnp.uint32)
    for p in range(8):
        sh = 28 - 4 * p
        jstar = jnp.zeros((batch, 1), jnp.uint32)
        for j in range(1, 16):
            cand = prefix | jnp.uint32(j << sh)
            cnt = jnp.sum(jnp.where(ukey >= cand, 1.0, 0.0), axis=1,
                          keepdims=True)
            jstar += jnp.where(cnt >= kf, jnp.uint32(1), jnp.uint32(0))
        prefix = prefix | jax.lax.shift_left(jstar, jnp.uint32(sh))
    return prefix


def _encoder_kernel(x_ref, wi_ref, wr_ref, out_ref, u_ref, *, steps, batch,
                    rec, kk):
    u_ref[:] = jnp.dot(x_ref[:], wi_ref[:], preferred_element_type=jnp.float32)
    wr = wr_ref[:]
    kf = jnp.float32(kk)

    def step(t, carry):
        a, inv_n = carry
        w = jnp.dot(a, wr, preferred_element_type=jnp.float32)
        bits = jax.lax.bitcast_convert_type(w, jnp.uint32)
        ukey = jnp.where(w < 0, ~bits, bits | jnp.uint32(0x80000000))
        prefix = _kth_threshold(ukey, kf, batch)
        s = jnp.where(ukey >= prefix, w * inv_n, 0.0)
        a_new = jnp.tanh(u_ref[pl.ds(t * batch, batch), :] + s)
        nrm = jnp.sqrt(jnp.sum(a_new * a_new, axis=1, keepdims=True))
        return a_new, 1.0 / (nrm + 1e-6)

    a0 = jnp.zeros((batch, rec), jnp.float32)
    inv0 = jnp.ones((batch, 1), jnp.float32)
    a_fin, inv_fin = jax.lax.fori_loop(0, steps, step, (a0, inv0),
                                       unroll=False)
    out_ref[:] = a_fin * inv_fin


def kernel(x, W_input, W_recurrent):
    B, T, H, W, E = x.shape
    R = T * H * W
    rec = W_recurrent.shape[0]
    kk = int(rec * 0.4)
    x2 = jnp.transpose(x.reshape(B, R, E), (1, 0, 2)).reshape(R * B, E)
    return pl.pallas_call(
        functools.partial(_encoder_kernel, steps=R, batch=B, rec=rec, kk=kk),
        out_shape=jax.ShapeDtypeStruct((B, rec), x.dtype),
        scratch_shapes=[pltpu.VMEM((R * B, rec), jnp.float32)],
    )(x2, W_input, W_recurrent)


# speedup vs baseline: 18.9400x; 1.0106x over previous
import functools

import jax
import jax.numpy as jnp
from jax.experimental import pallas as pl
from jax.experimental.pallas import tpu as pltpu


_PASS_BITS = (2, 3, 3, 3, 3, 3, 3, 3, 3, 3, 3)


def _kth_threshold(ukey, kf, batch):
    prefix = jnp.zeros((batch, 1), jnp.uint32)
    sh = 32
    for m in _PASS_BITS:
        sh -= m
        jstar = jnp.zeros((batch, 1), jnp.uint32)
        for j in range(1, 1 << m):
            cand = prefix | jnp.uint32(j << sh)
            cnt = jnp.sum(jnp.where(ukey >= cand, 1.0, 0.0), axis=1,
                          keepdims=True)
            jstar += jnp.where(cnt >= kf, jnp.uint32(1), jnp.uint32(0))
        prefix = prefix | jax.lax.shift_left(jstar, jnp.uint32(sh))
    return prefix


def _encoder_kernel(x_ref, wi_ref, wr_ref, out_ref, u_ref, *, steps, batch,
                    rec, kk):
    u_ref[:] = jnp.dot(x_ref[:], wi_ref[:], preferred_element_type=jnp.float32)
    wr = wr_ref[:]
    kf = jnp.float32(kk)

    def step(t, carry):
        a, inv_n = carry
        w = jnp.dot(a, wr, preferred_element_type=jnp.float32)
        bits = jax.lax.bitcast_convert_type(w, jnp.uint32)
        ukey = jnp.where(w < 0, ~bits, bits | jnp.uint32(0x80000000))
        prefix = _kth_threshold(ukey, kf, batch)
        s = jnp.where(ukey >= prefix, w * inv_n, 0.0)
        a_new = jnp.tanh(u_ref[pl.ds(t * batch, batch), :] + s)
        nrm = jnp.sqrt(jnp.sum(a_new * a_new, axis=1, keepdims=True))
        return a_new, 1.0 / (nrm + 1e-6)

    a0 = jnp.zeros((batch, rec), jnp.float32)
    inv0 = jnp.ones((batch, 1), jnp.float32)
    a_fin, inv_fin = jax.lax.fori_loop(0, steps, step, (a0, inv0),
                                       unroll=False)
    out_ref[:] = a_fin * inv_fin


def kernel(x, W_input, W_recurrent):
    B, T, H, W, E = x.shape
    R = T * H * W
    rec = W_recurrent.shape[0]
    kk = int(rec * 0.4)
    x2 = jnp.transpose(x.reshape(B, R, E), (1, 0, 2)).reshape(R * B, E)
    return pl.pallas_call(
        functools.partial(_encoder_kernel, steps=R, batch=B, rec=rec, kk=kk),
        out_shape=jax.ShapeDtypeStruct((B, rec), x.dtype),
        scratch_shapes=[pltpu.VMEM((R * B, rec), jnp.float32)],
    )(x2, W_input, W_recurrent)
